# Initial kernel scaffold; baseline (speedup 1.0000x reference)
import jax
import jax.numpy as jnp
from jax.experimental import pallas as pl


def kernel(X, A, W1, b1, W2, b2, Wout, bout):
    raise NotImplementedError("write your pallas kernel here")


# speedup vs baseline: 1.0273x; 1.0273x over previous
import jax
import jax.numpy as jnp
from jax.experimental import pallas as pl
from jax.experimental.pallas import tpu as pltpu

BM = 400


def _layer1_body(a_ref, x_ref, w1_ref, b1_ref, w2_ref, o_ref):
    acc = jnp.dot(a_ref[...], x_ref[...], preferred_element_type=jnp.float32)
    pre = jnp.dot(acc, w1_ref[...], preferred_element_type=jnp.float32) + b1_ref[...]
    h = jnp.where(pre > 0, pre, jnp.exp(pre) - 1.0)
    o_ref[...] = jnp.dot(h, w2_ref[...], preferred_element_type=jnp.float32)


def _layer2_body(a_ref, y_ref, b2_ref, wo_ref, bo_ref, o_ref):
    acc = jnp.dot(a_ref[...], y_ref[...], preferred_element_type=jnp.float32)
    pre = acc + b2_ref[...]
    h = jnp.where(pre > 0, pre, jnp.exp(pre) - 1.0)
    o_ref[...] = jnp.dot(h, wo_ref[...], preferred_element_type=jnp.float32) + bo_ref[...]


def kernel(X, A, W1, b1, W2, b2, Wout, bout):
    n, d_in = X.shape
    d_h = W1.shape[1]
    d_out = Wout.shape[1]
    grid = (n // BM,)

    b1r = b1.reshape(1, d_h)
    b2r = b2.reshape(1, d_h)
    boutr = bout.reshape(1, d_out)

    y2 = pl.pallas_call(
        _layer1_body,
        grid=grid,
        in_specs=[
            pl.BlockSpec((BM, n), lambda m: (m, 0)),
            pl.BlockSpec((n, d_in), lambda m: (0, 0)),
            pl.BlockSpec((d_in, d_h), lambda m: (0, 0)),
            pl.BlockSpec((1, d_h), lambda m: (0, 0)),
            pl.BlockSpec((d_h, d_h), lambda m: (0, 0)),
        ],
        out_specs=pl.BlockSpec((BM, d_h), lambda m: (m, 0)),
        out_shape=jax.ShapeDtypeStruct((n, d_h), jnp.float32),
        compiler_params=pltpu.CompilerParams(
            dimension_semantics=("arbitrary",)),
    )(A, X, W1, b1r, W2)

    logits = pl.pallas_call(
        _layer2_body,
        grid=grid,
        in_specs=[
            pl.BlockSpec((BM, n), lambda m: (m, 0)),
            pl.BlockSpec((n, d_h), lambda m: (0, 0)),
            pl.BlockSpec((1, d_h), lambda m: (0, 0)),
            pl.BlockSpec((d_h, d_out), lambda m: (0, 0)),
            pl.BlockSpec((1, d_out), lambda m: (0, 0)),
        ],
        out_specs=pl.BlockSpec((BM, d_out), lambda m: (m, 0)),
        out_shape=jax.ShapeDtypeStruct((n, d_out), jnp.float32),
        compiler_params=pltpu.CompilerParams(
            dimension_semantics=("arbitrary",)),
    )(A, y2, b2r, Wout, boutr)

    return logits
